# Initial kernel scaffold; baseline (speedup 1.0000x reference)
import functools

import jax
import jax.numpy as jnp
from jax import lax
from jax.experimental import pallas as pl
from jax.experimental.pallas import tpu as pltpu
from jax.experimental.pallas import tpu_sc as plsc

Q, K, D, C, KNN = 4096, 16384, 128, 21, 5
CONC = 16.0
QT = 128
BIG = jnp.int32(2**30)


def _topk_body(e_ref, p_ref, vals_ref, idx_ref, pn_ref):
    @pl.when(pl.program_id(0) == 0)
    def _():
        p = p_ref[...]
        pn_ref[...] = p / (jnp.sqrt(jnp.sum(p * p, axis=1, keepdims=True)) + 1e-12)

    e = e_ref[...]
    en = e / (jnp.sqrt(jnp.sum(e * e, axis=1, keepdims=True)) + 1e-12)
    sim = lax.dot_general(en, pn_ref[...], (((1,), (1,)), ((), ())),
                          preferred_element_type=jnp.float32) * CONC
    iota = lax.broadcasted_iota(jnp.int32, (QT, K), 1)
    m = jnp.full((QT, 1), jnp.inf, jnp.float32)
    a = jnp.full((QT, 1), -1, jnp.int32)
    for j in range(KNN):
        avail = (sim < m) | ((sim == m) & (iota > a))
        masked = jnp.where(avail, sim, -jnp.inf)
        m = jnp.max(masked, axis=1, keepdims=True)
        a = jnp.min(jnp.where(masked == m, iota, BIG), axis=1, keepdims=True)
        vals_ref[j, :] = m[:, 0]
        idx_ref[j, :] = a[:, 0]
    for j in range(KNN, 8):
        vals_ref[j, :] = jnp.zeros((QT,), jnp.float32)
        idx_ref[j, :] = jnp.zeros((QT,), jnp.int32)


_topk = pl.pallas_call(
    _topk_body,
    grid=(Q // QT,),
    in_specs=[
        pl.BlockSpec((QT, D), lambda i: (i, 0)),
        pl.BlockSpec((K, D), lambda i: (0, 0)),
    ],
    out_specs=[
        pl.BlockSpec((8, QT), lambda i: (0, i)),
        pl.BlockSpec((8, QT), lambda i: (0, i)),
    ],
    out_shape=[
        jax.ShapeDtypeStruct((8, Q), jnp.float32),
        jax.ShapeDtypeStruct((8, Q), jnp.int32),
    ],
    scratch_shapes=[pltpu.VMEM((K, D), jnp.float32)],
)

NW = 32
QPW = Q // NW
SCORES = QPW * C


@functools.partial(
    pl.kernel,
    mesh=plsc.VectorSubcoreMesh(core_axis_name="c", subcore_axis_name="s"),
    out_type=jax.ShapeDtypeStruct((Q * C,), jnp.float32),
    scratch_types=[
        pltpu.VMEM((K,), jnp.int32),
        pltpu.VMEM((8, QPW), jnp.float32),
        pltpu.VMEM((8, QPW), jnp.int32),
        pltpu.VMEM((SCORES,), jnp.float32),
    ],
)
def _vote(vals_hbm, idx_hbm, labels_hbm, out_hbm, labels_v, vals_v, idx_v, scores_v):
    wid = lax.axis_index("s") * 2 + lax.axis_index("c")
    base = wid * QPW
    pltpu.sync_copy(labels_hbm, labels_v)
    pltpu.sync_copy(vals_hbm.at[:, pl.ds(base, QPW)], vals_v)
    pltpu.sync_copy(idx_hbm.at[:, pl.ds(base, QPW)], idx_v)
    zeros = jnp.zeros((16,), jnp.float32)
    for g in range(SCORES // 16):
        scores_v[pl.ds(g * 16, 16)] = zeros
    for j in range(KNN):
        for g in range(QPW // 16):
            protos = idx_v[j, pl.ds(g * 16, 16)]
            v16 = vals_v[j, pl.ds(g * 16, 16)]
            labs = plsc.load_gather(labels_v, [protos])
            qloc = jnp.full((16,), g * 16, jnp.int32) + lax.iota(jnp.int32, 16)
            plsc.addupdate_scatter(scores_v, [qloc * C + labs], v16)
    pltpu.sync_copy(scores_v, out_hbm.at[pl.ds(base * C, SCORES)])


def kernel(embeddings, prototypes, prototype_labels):
    vals8, idx8 = _topk(embeddings, prototypes)
    flat = _vote(vals8, idx8, prototype_labels)
    return flat.reshape(Q, C)


# speedup vs baseline: 3.7615x; 3.7615x over previous
import functools

import jax
import jax.numpy as jnp
from jax import lax
from jax.experimental import pallas as pl
from jax.experimental.pallas import tpu as pltpu
from jax.experimental.pallas import tpu_sc as plsc

Q, K, D, C, KNN = 4096, 16384, 128, 21, 5
CONC = 16.0
QT = 128
BIG = 2**30


def _topk_body(e_ref, p_ref, vals_ref, idx_ref, pn_ref):
    @pl.when(pl.program_id(0) == 0)
    def _():
        p = p_ref[...]
        pn_ref[...] = p / (jnp.sqrt(jnp.sum(p * p, axis=1, keepdims=True)) + 1e-12)

    e = e_ref[...]
    en = e / (jnp.sqrt(jnp.sum(e * e, axis=1, keepdims=True)) + 1e-12)
    sim = lax.dot_general(en, pn_ref[...], (((1,), (1,)), ((), ())),
                          preferred_element_type=jnp.float32) * CONC
    iota = lax.broadcasted_iota(jnp.int32, (QT, K), 1)
    m = jnp.full((QT, 1), jnp.inf, jnp.float32)
    a = jnp.full((QT, 1), -1, jnp.int32)
    for j in range(KNN):
        avail = (sim < m) | ((sim == m) & (iota > a))
        masked = jnp.where(avail, sim, -jnp.inf)
        m = jnp.max(masked, axis=1, keepdims=True)
        a = jnp.min(jnp.where(masked == m, iota, BIG), axis=1, keepdims=True)
        vals_ref[j, :] = m[:, 0]
        idx_ref[j, :] = a[:, 0]
    for j in range(KNN, 8):
        vals_ref[j, :] = jnp.zeros((QT,), jnp.float32)
        idx_ref[j, :] = jnp.zeros((QT,), jnp.int32)


_topk = pl.pallas_call(
    _topk_body,
    grid=(Q // QT,),
    in_specs=[
        pl.BlockSpec((QT, D), lambda i: (i, 0)),
        pl.BlockSpec((K, D), lambda i: (0, 0)),
    ],
    out_specs=[
        pl.BlockSpec((8, QT), lambda i: (0, i)),
        pl.BlockSpec((8, QT), lambda i: (0, i)),
    ],
    out_shape=[
        jax.ShapeDtypeStruct((8, Q), jnp.float32),
        jax.ShapeDtypeStruct((8, Q), jnp.int32),
    ],
    scratch_shapes=[pltpu.VMEM((K, D), jnp.float32)],
)

NW = 32
QPW = Q // NW
SCORES = QPW * C


@functools.cache
def _make_vote():
    @functools.partial(
        pl.kernel,
        mesh=plsc.VectorSubcoreMesh(core_axis_name="c", subcore_axis_name="s"),
        compiler_params=pltpu.CompilerParams(needs_layout_passes=False),
        out_type=jax.ShapeDtypeStruct((Q * C,), jnp.float32),
        scratch_types=[
            pltpu.VMEM((K,), jnp.int32),
            pltpu.VMEM((8, QPW), jnp.float32),
            pltpu.VMEM((8, QPW), jnp.int32),
            pltpu.VMEM((SCORES,), jnp.float32),
        ],
    )
    def _vote(vals_hbm, idx_hbm, labels_hbm, out_hbm,
              labels_v, vals_v, idx_v, scores_v):
        wid = lax.axis_index("s") * 2 + lax.axis_index("c")
        base = wid * QPW
        pltpu.sync_copy(labels_hbm, labels_v)
        pltpu.sync_copy(vals_hbm.at[:, pl.ds(base, QPW)], vals_v)
        pltpu.sync_copy(idx_hbm.at[:, pl.ds(base, QPW)], idx_v)
        zeros = jnp.zeros((16,), jnp.float32)
        for g in range(SCORES // 16):
            scores_v[pl.ds(g * 16, 16)] = zeros
        for j in range(KNN):
            for g in range(QPW // 16):
                protos = idx_v[j, pl.ds(g * 16, 16)]
                v16 = vals_v[j, pl.ds(g * 16, 16)]
                labs = plsc.load_gather(labels_v, [protos])
                qloc = jnp.full((16,), g * 16, jnp.int32) + lax.iota(jnp.int32, 16)
                plsc.addupdate_scatter(scores_v, [qloc * C + labs], v16)
        pltpu.sync_copy(scores_v, out_hbm.at[pl.ds(base * C, SCORES)])

    return _vote


def kernel(embeddings, prototypes, prototype_labels):
    vals8, idx8 = _topk(embeddings, prototypes)
    flat = _make_vote()(vals8, idx8, prototype_labels)
    return flat.reshape(Q, C)


# speedup vs baseline: 5.6308x; 1.4969x over previous
import functools

import jax
import jax.numpy as jnp
from jax import lax
from jax.experimental import pallas as pl
from jax.experimental.pallas import tpu as pltpu
from jax.experimental.pallas import tpu_sc as plsc

Q, K, D, C, KNN = 4096, 16384, 128, 21, 5
CONC = 16.0
QT = 128
BIG = 2**30


def _topk_body(e_ref, p_ref, vals_ref, idx_ref, pn_ref):
    @pl.when(pl.program_id(0) == 0)
    def _():
        p = p_ref[...]
        pn_ref[...] = p / (jnp.sqrt(jnp.sum(p * p, axis=1, keepdims=True)) + 1e-12)

    e = e_ref[...]
    en = e / (jnp.sqrt(jnp.sum(e * e, axis=1, keepdims=True)) + 1e-12)
    sim = lax.dot_general(en, pn_ref[...], (((1,), (1,)), ((), ())),
                          preferred_element_type=jnp.float32) * CONC
    iota = lax.broadcasted_iota(jnp.int32, (QT, K), 1)
    masked = sim
    for j in range(KNN):
        m = jnp.max(masked, axis=1, keepdims=True)
        eqm = masked == m
        a = jnp.min(jnp.where(eqm, iota, BIG), axis=1, keepdims=True)
        vals_ref[j, :] = m[:, 0]
        idx_ref[j, :] = a[:, 0]
        if j < KNN - 1:
            masked = jnp.where(iota == a, -jnp.inf, masked)
    for j in range(KNN, 8):
        vals_ref[j, :] = jnp.zeros((QT,), jnp.float32)
        idx_ref[j, :] = jnp.zeros((QT,), jnp.int32)


_topk = pl.pallas_call(
    _topk_body,
    grid=(Q // QT,),
    in_specs=[
        pl.BlockSpec((QT, D), lambda i: (i, 0)),
        pl.BlockSpec((K, D), lambda i: (0, 0)),
    ],
    out_specs=[
        pl.BlockSpec((8, QT), lambda i: (0, i)),
        pl.BlockSpec((8, QT), lambda i: (0, i)),
    ],
    out_shape=[
        jax.ShapeDtypeStruct((8, Q), jnp.float32),
        jax.ShapeDtypeStruct((8, Q), jnp.int32),
    ],
    scratch_shapes=[pltpu.VMEM((K, D), jnp.float32)],
)

NW = 32
QPW = Q // NW
SCORES = QPW * C


@functools.cache
def _make_vote():
    @functools.partial(
        pl.kernel,
        mesh=plsc.VectorSubcoreMesh(core_axis_name="c", subcore_axis_name="s"),
        compiler_params=pltpu.CompilerParams(needs_layout_passes=False),
        out_type=jax.ShapeDtypeStruct((Q * C,), jnp.float32),
        scratch_types=[
            pltpu.VMEM((K,), jnp.int32),
            pltpu.VMEM((8, QPW), jnp.float32),
            pltpu.VMEM((8, QPW), jnp.int32),
            pltpu.VMEM((SCORES,), jnp.float32),
        ],
    )
    def _vote(vals_hbm, idx_hbm, labels_hbm, out_hbm,
              labels_v, vals_v, idx_v, scores_v):
        wid = lax.axis_index("s") * 2 + lax.axis_index("c")
        base = wid * QPW
        pltpu.sync_copy(labels_hbm, labels_v)
        pltpu.sync_copy(vals_hbm.at[:, pl.ds(base, QPW)], vals_v)
        pltpu.sync_copy(idx_hbm.at[:, pl.ds(base, QPW)], idx_v)
        zeros = jnp.zeros((16,), jnp.float32)
        for g in range(SCORES // 16):
            scores_v[pl.ds(g * 16, 16)] = zeros
        for j in range(KNN):
            for g in range(QPW // 16):
                protos = idx_v[j, pl.ds(g * 16, 16)]
                v16 = vals_v[j, pl.ds(g * 16, 16)]
                labs = plsc.load_gather(labels_v, [protos])
                qloc = jnp.full((16,), g * 16, jnp.int32) + lax.iota(jnp.int32, 16)
                plsc.addupdate_scatter(scores_v, [qloc * C + labs], v16)
        pltpu.sync_copy(scores_v, out_hbm.at[pl.ds(base * C, SCORES)])

    return _vote


def kernel(embeddings, prototypes, prototype_labels):
    vals8, idx8 = _topk(embeddings, prototypes)
    flat = _make_vote()(vals8, idx8, prototype_labels)
    return flat.reshape(Q, C)


# speedup vs baseline: 5.8604x; 1.0408x over previous
import functools

import jax
import jax.numpy as jnp
from jax import lax
from jax.experimental import pallas as pl
from jax.experimental.pallas import tpu as pltpu
from jax.experimental.pallas import tpu_sc as plsc

Q, K, D, C, KNN = 4096, 16384, 128, 21, 5
CONC = 16.0
QT = 128
BIG = 2**30


def _pnorm_body(p_ref, pn_ref):
    p = p_ref[...]
    rp = 1.0 / (jnp.sqrt(jnp.sum(p * p, axis=1, keepdims=True)) + 1e-12)
    pn_ref[...] = p * rp


_pnorm = pl.pallas_call(
    _pnorm_body,
    out_shape=jax.ShapeDtypeStruct((K, D), jnp.float32),
)


def _topk_body(e_ref, pn_ref, vals_ref, idx_ref):
    e = e_ref[...]
    re_ = 1.0 / (jnp.sqrt(jnp.sum(e * e, axis=1, keepdims=True)) + 1e-12)
    en = e * (re_ * CONC)
    sim = lax.dot_general(en, pn_ref[...], (((1,), (1,)), ((), ())),
                          preferred_element_type=jnp.float32)
    H = K // 2
    iL = lax.broadcasted_iota(jnp.int32, (QT, H), 1)
    left, right = sim[:, :H], sim[:, H:]
    pmax = jnp.maximum(left, right)
    pmin = jnp.minimum(left, right)
    imax = jnp.where(left >= right, iL, iL + H)
    imin = (2 * iL + H) - imax
    for j in range(KNN):
        m = jnp.max(pmax, axis=1, keepdims=True)
        a = jnp.min(jnp.where(pmax == m, imax, BIG), axis=1, keepdims=True)
        vals_ref[j, :] = m[:, 0]
        idx_ref[j, :] = a[:, 0]
        if j < KNN - 1:
            pmax = jnp.where(imax == a, pmin, pmax)
            new_imax = jnp.where(imax == a, imin, imax)
            pmin = jnp.where(imax == a, -jnp.inf, pmin)
            imax = new_imax
    for j in range(KNN, 8):
        vals_ref[j, :] = jnp.zeros((QT,), jnp.float32)
        idx_ref[j, :] = jnp.zeros((QT,), jnp.int32)


_topk = pl.pallas_call(
    _topk_body,
    grid=(Q // QT,),
    in_specs=[
        pl.BlockSpec((QT, D), lambda i: (i, 0)),
        pl.BlockSpec((K, D), lambda i: (0, 0)),
    ],
    out_specs=[
        pl.BlockSpec((8, QT), lambda i: (0, i)),
        pl.BlockSpec((8, QT), lambda i: (0, i)),
    ],
    out_shape=[
        jax.ShapeDtypeStruct((8, Q), jnp.float32),
        jax.ShapeDtypeStruct((8, Q), jnp.int32),
    ],
)

NW = 32
QPW = Q // NW
SCORES = QPW * C


@functools.cache
def _make_vote():
    @functools.partial(
        pl.kernel,
        mesh=plsc.VectorSubcoreMesh(core_axis_name="c", subcore_axis_name="s"),
        compiler_params=pltpu.CompilerParams(needs_layout_passes=False),
        out_type=jax.ShapeDtypeStruct((Q * C,), jnp.float32),
        scratch_types=[
            pltpu.VMEM((K,), jnp.int32),
            pltpu.VMEM((8, QPW), jnp.float32),
            pltpu.VMEM((8, QPW), jnp.int32),
            pltpu.VMEM((SCORES,), jnp.float32),
        ],
    )
    def _vote(vals_hbm, idx_hbm, labels_hbm, out_hbm,
              labels_v, vals_v, idx_v, scores_v):
        wid = lax.axis_index("s") * 2 + lax.axis_index("c")
        base = wid * QPW
        pltpu.sync_copy(labels_hbm, labels_v)
        pltpu.sync_copy(vals_hbm.at[:, pl.ds(base, QPW)], vals_v)
        pltpu.sync_copy(idx_hbm.at[:, pl.ds(base, QPW)], idx_v)
        zeros = jnp.zeros((16,), jnp.float32)
        for g in range(SCORES // 16):
            scores_v[pl.ds(g * 16, 16)] = zeros
        for j in range(KNN):
            for g in range(QPW // 16):
                protos = idx_v[j, pl.ds(g * 16, 16)]
                v16 = vals_v[j, pl.ds(g * 16, 16)]
                labs = plsc.load_gather(labels_v, [protos])
                qloc = jnp.full((16,), g * 16, jnp.int32) + lax.iota(jnp.int32, 16)
                plsc.addupdate_scatter(scores_v, [qloc * C + labs], v16)
        pltpu.sync_copy(scores_v, out_hbm.at[pl.ds(base * C, SCORES)])

    return _vote


def kernel(embeddings, prototypes, prototype_labels):
    pn = _pnorm(prototypes)
    vals8, idx8 = _topk(embeddings, pn)
    flat = _make_vote()(vals8, idx8, prototype_labels)
    return flat.reshape(Q, C)


# speedup vs baseline: 5.9624x; 1.0174x over previous
import functools

import jax
import jax.numpy as jnp
from jax import lax
from jax.experimental import pallas as pl
from jax.experimental.pallas import tpu as pltpu
from jax.experimental.pallas import tpu_sc as plsc

Q, K, D, C, KNN = 4096, 16384, 128, 21, 5
CONC = 16.0
QT = 128
BIG = 2**30


NT = Q // QT


def _topk_body(e_ref, p_ref, vals_ref, idx_ref, pn_ref):
    @pl.when(pl.program_id(0) == 0)
    def _():
        p = p_ref[...]
        rp = 1.0 / (jnp.sqrt(jnp.sum(p * p, axis=1, keepdims=True)) + 1e-12)
        pn_ref[...] = p * rp

    e = e_ref[...]
    re_ = 1.0 / (jnp.sqrt(jnp.sum(e * e, axis=1, keepdims=True)) + 1e-12)
    en = e * (re_ * CONC)
    sim = lax.dot_general(en, pn_ref[...], (((1,), (1,)), ((), ())),
                          preferred_element_type=jnp.float32)
    H = K // 2
    iL = lax.broadcasted_iota(jnp.int32, (QT, H), 1)
    left, right = sim[:, :H], sim[:, H:]
    pmax = jnp.maximum(left, right)
    pmin = jnp.minimum(left, right)
    imax = jnp.where(left >= right, iL, iL + H)
    imin = (2 * iL + H) - imax
    for j in range(KNN):
        m = jnp.max(pmax, axis=1, keepdims=True)
        a = jnp.min(jnp.where(pmax == m, imax, BIG), axis=1, keepdims=True)
        vals_ref[j, :] = m[:, 0]
        idx_ref[j, :] = a[:, 0]
        if j < KNN - 1:
            pmax = jnp.where(imax == a, pmin, pmax)
            new_imax = jnp.where(imax == a, imin, imax)
            pmin = jnp.where(imax == a, -jnp.inf, pmin)
            imax = new_imax
    for j in range(KNN, 8):
        vals_ref[j, :] = jnp.zeros((QT,), jnp.float32)
        idx_ref[j, :] = jnp.zeros((QT,), jnp.int32)


_topk = pl.pallas_call(
    _topk_body,
    grid=(NT,),
    in_specs=[
        pl.BlockSpec((QT, D), lambda i: (i, 0)),
        pl.BlockSpec((K, D), lambda i: (0, 0)),
    ],
    out_specs=[
        pl.BlockSpec((8, QT), lambda i: (0, i)),
        pl.BlockSpec((8, QT), lambda i: (0, i)),
    ],
    out_shape=[
        jax.ShapeDtypeStruct((8, Q), jnp.float32),
        jax.ShapeDtypeStruct((8, Q), jnp.int32),
    ],
    scratch_shapes=[pltpu.VMEM((K, D), jnp.float32)],
)

NW = 32
QPW = Q // NW
SCORES = QPW * C


@functools.cache
def _make_vote():
    @functools.partial(
        pl.kernel,
        mesh=plsc.VectorSubcoreMesh(core_axis_name="c", subcore_axis_name="s"),
        compiler_params=pltpu.CompilerParams(needs_layout_passes=False),
        out_type=jax.ShapeDtypeStruct((Q * C,), jnp.float32),
        scratch_types=[
            pltpu.VMEM((K,), jnp.int32),
            pltpu.VMEM((8, QPW), jnp.float32),
            pltpu.VMEM((8, QPW), jnp.int32),
            pltpu.VMEM((SCORES,), jnp.float32),
        ],
    )
    def _vote(vals_hbm, idx_hbm, labels_hbm, out_hbm,
              labels_v, vals_v, idx_v, scores_v):
        wid = lax.axis_index("s") * 2 + lax.axis_index("c")
        base = wid * QPW
        pltpu.sync_copy(labels_hbm, labels_v)
        pltpu.sync_copy(vals_hbm.at[:, pl.ds(base, QPW)], vals_v)
        pltpu.sync_copy(idx_hbm.at[:, pl.ds(base, QPW)], idx_v)
        zeros = jnp.zeros((16,), jnp.float32)
        for g in range(SCORES // 16):
            scores_v[pl.ds(g * 16, 16)] = zeros
        for j in range(KNN):
            for g in range(QPW // 16):
                protos = idx_v[j, pl.ds(g * 16, 16)]
                v16 = vals_v[j, pl.ds(g * 16, 16)]
                labs = plsc.load_gather(labels_v, [protos])
                qloc = jnp.full((16,), g * 16, jnp.int32) + lax.iota(jnp.int32, 16)
                plsc.addupdate_scatter(scores_v, [qloc * C + labs], v16)
        pltpu.sync_copy(scores_v, out_hbm.at[pl.ds(base * C, SCORES)])

    return _vote


def kernel(embeddings, prototypes, prototype_labels):
    vals8, idx8 = _topk(embeddings, prototypes)
    flat = _make_vote()(vals8, idx8, prototype_labels)
    return flat.reshape(Q, C)


# speedup vs baseline: 5.9627x; 1.0000x over previous
import functools

import jax
import jax.numpy as jnp
from jax import lax
from jax.experimental import pallas as pl
from jax.experimental.pallas import tpu as pltpu
from jax.experimental.pallas import tpu_sc as plsc

Q, K, D, C, KNN = 4096, 16384, 128, 21, 5
CONC = 16.0
QT = 128
BIG = 2**30


NT = Q // QT


def _topk_body(e_ref, p_ref, vals_ref, idx_ref, pn_ref):
    @pl.when(pl.program_id(0) == 0)
    def _():
        p = p_ref[...]
        rp = 1.0 / (jnp.sqrt(jnp.sum(p * p, axis=1, keepdims=True)) + 1e-12)
        pn_ref[...] = p * rp

    e = e_ref[...]
    re_ = 1.0 / (jnp.sqrt(jnp.sum(e * e, axis=1, keepdims=True)) + 1e-12)
    en = e * (re_ * CONC)
    sim = lax.dot_general(en, pn_ref[...], (((1,), (1,)), ((), ())),
                          preferred_element_type=jnp.float32)
    H = K // 2
    iL = lax.broadcasted_iota(jnp.int32, (QT, H), 1)
    left, right = sim[:, :H], sim[:, H:]
    pmax = jnp.maximum(left, right)
    pmin = jnp.minimum(left, right)
    imax = jnp.where(left >= right, iL, iL + H)
    imin = (2 * iL + H) - imax
    for j in range(KNN):
        m = jnp.max(pmax, axis=1, keepdims=True)
        a = jnp.min(jnp.where(pmax == m, imax, BIG), axis=1, keepdims=True)
        vals_ref[j, :] = m[:, 0]
        idx_ref[j, :] = a[:, 0]
        if j < KNN - 1:
            sel = imax == a
            pmax = jnp.where(sel, pmin, pmax)
            imax = jnp.where(sel, imin, imax)
            pmin = jnp.where(sel, -jnp.inf, pmin)
    for j in range(KNN, 8):
        vals_ref[j, :] = jnp.zeros((QT,), jnp.float32)
        idx_ref[j, :] = jnp.zeros((QT,), jnp.int32)


_topk = pl.pallas_call(
    _topk_body,
    grid=(NT,),
    in_specs=[
        pl.BlockSpec((QT, D), lambda i: (i, 0)),
        pl.BlockSpec((K, D), lambda i: (0, 0)),
    ],
    out_specs=[
        pl.BlockSpec((8, QT), lambda i: (0, i)),
        pl.BlockSpec((8, QT), lambda i: (0, i)),
    ],
    out_shape=[
        jax.ShapeDtypeStruct((8, Q), jnp.float32),
        jax.ShapeDtypeStruct((8, Q), jnp.int32),
    ],
    scratch_shapes=[pltpu.VMEM((K, D), jnp.float32)],
)

NW = 32
QPW = Q // NW
SCORES = QPW * C


@functools.cache
def _make_vote():
    @functools.partial(
        pl.kernel,
        mesh=plsc.VectorSubcoreMesh(core_axis_name="c", subcore_axis_name="s"),
        compiler_params=pltpu.CompilerParams(needs_layout_passes=False),
        out_type=jax.ShapeDtypeStruct((Q * C,), jnp.float32),
        scratch_types=[
            pltpu.VMEM((K,), jnp.int32),
            pltpu.VMEM((8, QPW), jnp.float32),
            pltpu.VMEM((8, QPW), jnp.int32),
            pltpu.VMEM((SCORES,), jnp.float32),
        ],
    )
    def _vote(vals_hbm, idx_hbm, labels_hbm, out_hbm,
              labels_v, vals_v, idx_v, scores_v):
        wid = lax.axis_index("s") * 2 + lax.axis_index("c")
        base = wid * QPW
        pltpu.sync_copy(labels_hbm, labels_v)
        pltpu.sync_copy(vals_hbm.at[:, pl.ds(base, QPW)], vals_v)
        pltpu.sync_copy(idx_hbm.at[:, pl.ds(base, QPW)], idx_v)
        zeros = jnp.zeros((16,), jnp.float32)
        for g in range(SCORES // 16):
            scores_v[pl.ds(g * 16, 16)] = zeros
        for j in range(KNN):
            for g in range(QPW // 16):
                protos = idx_v[j, pl.ds(g * 16, 16)]
                v16 = vals_v[j, pl.ds(g * 16, 16)]
                labs = plsc.load_gather(labels_v, [protos])
                qloc = jnp.full((16,), g * 16, jnp.int32) + lax.iota(jnp.int32, 16)
                plsc.addupdate_scatter(scores_v, [qloc * C + labs], v16)
        pltpu.sync_copy(scores_v, out_hbm.at[pl.ds(base * C, SCORES)])

    return _vote


def kernel(embeddings, prototypes, prototype_labels):
    vals8, idx8 = _topk(embeddings, prototypes)
    flat = _make_vote()(vals8, idx8, prototype_labels)
    return flat.reshape(Q, C)
